# baseline (speedup 1.0000x reference)
import math

import jax
import jax.numpy as jnp
import numpy as np
from jax import lax
from jax.experimental import pallas as pl
from jax.experimental.pallas import tpu as pltpu
from jax.experimental.pallas import tpu_sc as plsc

DICT_SIZE = 1000000
D = 64
L_SEQ = 200
B = 1024
NW = 32
ROWS_PER_W = B // NW
CHUNK = 40
NCHUNK = L_SEQ // CHUNK
LANES = 16
NVREG_ROW = D // LANES
SCALE = math.sqrt(D)


def _positional_encoding_np(seq_len, d_model):
    pos = np.arange(seq_len, dtype=np.float32)[:, None]
    div = np.exp(
        np.arange(0, d_model, 2, dtype=np.float32)
        * (-math.log(10000.0) / d_model)
    )
    pe = np.zeros((seq_len, d_model), dtype=np.float32)
    pe[:, 0::2] = np.sin(pos * div)
    pe[:, 1::2] = np.cos(pos * div)
    return pe


_PE = _positional_encoding_np(L_SEQ, D)


def _sc_body(x_hbm, pe_hbm, emb_hbm, out_hbm, idx_v, pe_v, buf_v, sem):
    c = lax.axis_index("c")
    s = lax.axis_index("s")
    w = s * 2 + c
    base = w * (ROWS_PER_W * L_SEQ)

    pltpu.sync_copy(x_hbm.at[pl.ds(base, ROWS_PER_W * L_SEQ)], idx_v)
    pltpu.sync_copy(pe_hbm, pe_v)

    def do_row(r):
        copies = []
        for ch in range(NCHUNK):
            copies.append(
                pltpu.async_copy(
                    emb_hbm.at[idx_v.at[pl.ds(r * L_SEQ + ch * CHUNK, CHUNK)]],
                    buf_v.at[pl.ds(ch * CHUNK, CHUNK)],
                    sem,
                )
            )
        for cp in copies:
            cp.wait()

        def fma_row(j, carry):
            for k in range(NVREG_ROW):
                sl = pl.ds(k * LANES, LANES)
                buf_v[j, sl] = buf_v[j, sl] * SCALE + pe_v[j, sl]
            return carry

        lax.fori_loop(0, L_SEQ, fma_row, 0)

        pltpu.sync_copy(buf_v, out_hbm.at[pl.ds(base + r * L_SEQ, L_SEQ)])

    def row_loop(r, carry):
        do_row(r)
        return carry

    lax.fori_loop(0, ROWS_PER_W, row_loop, 0)


@jax.jit
def _pre_layer_sc(x_flat, pe, emb_weight):
    mesh = plsc.VectorSubcoreMesh(core_axis_name="c", subcore_axis_name="s")
    k = pl.kernel(
        _sc_body,
        out_type=jax.ShapeDtypeStruct((B * L_SEQ, D), jnp.float32),
        mesh=mesh,
        scratch_types=[
            pltpu.VMEM((ROWS_PER_W * L_SEQ,), jnp.int32),
            pltpu.VMEM((L_SEQ, D), jnp.float32),
            pltpu.VMEM((L_SEQ, D), jnp.float32),
            pltpu.SemaphoreType.DMA,
        ],
        compiler_params=pltpu.CompilerParams(use_tc_tiling_on_sc=False),
    )
    return k(x_flat, pe, emb_weight)


def kernel(x, emb_weight):
    x_flat = x.reshape(B * L_SEQ).astype(jnp.int32)
    pe = jnp.asarray(_PE)
    out = _pre_layer_sc(x_flat, pe, emb_weight)
    return out.reshape(B, L_SEQ, D)


# speedup vs baseline: 1.0504x; 1.0504x over previous
import math

import jax
import jax.numpy as jnp
import numpy as np
from jax import lax
from jax.experimental import pallas as pl
from jax.experimental.pallas import tpu as pltpu
from jax.experimental.pallas import tpu_sc as plsc

DICT_SIZE = 1000000
D = 64
L_SEQ = 200
B = 1024
NW = 32
ROWS_PER_W = B // NW
CHUNK = 40
NCHUNK = L_SEQ // CHUNK
LANES = 16
NVREG_ROW = D // LANES
NBUF = 4
NROUND = ROWS_PER_W // NBUF
SCALE = math.sqrt(D)


def _positional_encoding_np(seq_len, d_model):
    pos = np.arange(seq_len, dtype=np.float32)[:, None]
    div = np.exp(
        np.arange(0, d_model, 2, dtype=np.float32)
        * (-math.log(10000.0) / d_model)
    )
    pe = np.zeros((seq_len, d_model), dtype=np.float32)
    pe[:, 0::2] = np.sin(pos * div)
    pe[:, 1::2] = np.cos(pos * div)
    return pe


_PE = _positional_encoding_np(L_SEQ, D)


def _sc_body(x_hbm, pe_hbm, emb_hbm, out_hbm, idx_v, pe_v, bufs, gsem, wsem):
    c = lax.axis_index("c")
    s = lax.axis_index("s")
    w = s * 2 + c
    base = w * (ROWS_PER_W * L_SEQ)

    pltpu.sync_copy(x_hbm.at[pl.ds(base, ROWS_PER_W * L_SEQ)], idx_v)
    pltpu.sync_copy(pe_hbm, pe_v)

    def gather_row(r, b):
        for ch in range(NCHUNK):
            pltpu.async_copy(
                emb_hbm.at[idx_v.at[pl.ds(r * L_SEQ + ch * CHUNK, CHUNK)]],
                bufs.at[b, pl.ds(ch * CHUNK, CHUNK)],
                gsem.at[b],
            )

    def wait_gather(b):
        pltpu.make_async_copy(
            emb_hbm.at[pl.ds(0, L_SEQ)], bufs.at[b], gsem.at[b]
        ).wait()

    def wb_row(r, b):
        pltpu.async_copy(
            bufs.at[b], out_hbm.at[pl.ds(base + r * L_SEQ, L_SEQ)], wsem.at[b]
        )

    def wait_wb(b):
        pltpu.make_async_copy(
            bufs.at[b], out_hbm.at[pl.ds(base, L_SEQ)], wsem.at[b]
        ).wait()

    def compute(b):
        @plsc.parallel_loop(0, L_SEQ, unroll=8)
        def _(j):
            for k in range(NVREG_ROW):
                sl = pl.ds(k * LANES, LANES)
                bufs[b, j, sl] = bufs[b, j, sl] * SCALE + pe_v[j, sl]

    for b in range(NBUF):
        gather_row(b, b)

    @pl.loop(0, NROUND)
    def _(g):
        r0 = g * NBUF
        for b in range(NBUF):
            wait_gather(b)
            compute(b)
            wb_row(r0 + b, b)
        @pl.when(g < NROUND - 1)
        def _():
            for b in range(NBUF):
                wait_wb(b)
                gather_row(r0 + NBUF + b, b)

    for b in range(NBUF):
        wait_wb(b)


@jax.jit
def _pre_layer_sc(x_flat, pe, emb_weight):
    mesh = plsc.VectorSubcoreMesh(core_axis_name="c", subcore_axis_name="s")
    k = pl.kernel(
        _sc_body,
        out_type=jax.ShapeDtypeStruct((B * L_SEQ, D), jnp.float32),
        mesh=mesh,
        scratch_types=[
            pltpu.VMEM((ROWS_PER_W * L_SEQ,), jnp.int32),
            pltpu.VMEM((L_SEQ, D), jnp.float32),
            pltpu.VMEM((NBUF, L_SEQ, D), jnp.float32),
            pltpu.SemaphoreType.DMA((NBUF,)),
            pltpu.SemaphoreType.DMA((NBUF,)),
        ],
        compiler_params=pltpu.CompilerParams(use_tc_tiling_on_sc=False),
    )
    return k(x_flat, pe, emb_weight)


def kernel(x, emb_weight):
    x_flat = x.reshape(B * L_SEQ).astype(jnp.int32)
    pe = jnp.asarray(_PE)
    out = _pre_layer_sc(x_flat, pe, emb_weight)
    return out.reshape(B, L_SEQ, D)


# speedup vs baseline: 1.0534x; 1.0029x over previous
import math

import jax
import jax.numpy as jnp
import numpy as np
from jax import lax
from jax.experimental import pallas as pl
from jax.experimental.pallas import tpu as pltpu
from jax.experimental.pallas import tpu_sc as plsc

DICT_SIZE = 1000000
D = 64
L_SEQ = 200
B = 1024
NW = 32
ROWS_PER_W = B // NW
CHUNK = 40
NCHUNK = L_SEQ // CHUNK
LANES = 16
NVREG_ROW = D // LANES
NBUF = 4
NROUND = ROWS_PER_W // NBUF
SCALE = math.sqrt(D)


def _positional_encoding_np(seq_len, d_model):
    pos = np.arange(seq_len, dtype=np.float32)[:, None]
    div = np.exp(
        np.arange(0, d_model, 2, dtype=np.float32)
        * (-math.log(10000.0) / d_model)
    )
    pe = np.zeros((seq_len, d_model), dtype=np.float32)
    pe[:, 0::2] = np.sin(pos * div)
    pe[:, 1::2] = np.cos(pos * div)
    return pe


_PE = _positional_encoding_np(L_SEQ, D)


def _sc_body(x_hbm, pe_hbm, emb_hbm, out_hbm, idx_v, pe_v, bufs, gsem, wsem):
    c = lax.axis_index("c")
    s = lax.axis_index("s")
    w = s * 2 + c
    row0 = w * ROWS_PER_W

    pltpu.sync_copy(x_hbm.at[pl.ds(row0, ROWS_PER_W)], idx_v)
    pltpu.sync_copy(pe_hbm, pe_v)

    def gather_row(r, b):
        for ch in range(NCHUNK):
            pltpu.async_copy(
                emb_hbm.at[idx_v.at[r, pl.ds(ch * CHUNK, CHUNK)]],
                bufs.at[b, pl.ds(ch * CHUNK, CHUNK)],
                gsem.at[b],
            )

    def wait_gather(b):
        pltpu.make_async_copy(
            emb_hbm.at[pl.ds(0, L_SEQ)], bufs.at[b], gsem.at[b]
        ).wait()

    def wb_row(r, b):
        pltpu.async_copy(bufs.at[b], out_hbm.at[row0 + r], wsem.at[b])

    def wait_wb(b):
        pltpu.make_async_copy(
            bufs.at[b], out_hbm.at[row0], wsem.at[b]
        ).wait()

    def compute(b):
        @plsc.parallel_loop(0, L_SEQ, unroll=8)
        def _(j):
            for k in range(NVREG_ROW):
                sl = pl.ds(k * LANES, LANES)
                bufs[b, j, sl] = bufs[b, j, sl] * SCALE + pe_v[j, sl]

    for b in range(NBUF):
        gather_row(b, b)

    @pl.loop(0, NROUND)
    def _(g):
        r0 = g * NBUF
        for b in range(NBUF):
            wait_gather(b)
            compute(b)
            wb_row(r0 + b, b)
        @pl.when(g < NROUND - 1)
        def _():
            for b in range(NBUF):
                wait_wb(b)
                gather_row(r0 + NBUF + b, b)

    for b in range(NBUF):
        wait_wb(b)


@jax.jit
def _pre_layer_sc(x, pe, emb_weight):
    mesh = plsc.VectorSubcoreMesh(core_axis_name="c", subcore_axis_name="s")
    k = pl.kernel(
        _sc_body,
        out_type=jax.ShapeDtypeStruct((B, L_SEQ, D), jnp.float32),
        mesh=mesh,
        scratch_types=[
            pltpu.VMEM((ROWS_PER_W, L_SEQ), jnp.int32),
            pltpu.VMEM((L_SEQ, D), jnp.float32),
            pltpu.VMEM((NBUF, L_SEQ, D), jnp.float32),
            pltpu.SemaphoreType.DMA((NBUF,)),
            pltpu.SemaphoreType.DMA((NBUF,)),
        ],
        compiler_params=pltpu.CompilerParams(use_tc_tiling_on_sc=False),
    )
    return k(x, pe, emb_weight)


def kernel(x, emb_weight):
    pe = jnp.asarray(_PE)
    return _pre_layer_sc(x.astype(jnp.int32), pe, emb_weight)
